# Initial kernel scaffold; baseline (speedup 1.0000x reference)
import jax
import jax.numpy as jnp
from jax.experimental import pallas as pl


def kernel(x, W_in, Wl1, Wr1, b1, Wl2, Wr2, b2, W_out, b_out):
    raise NotImplementedError("write your pallas kernel here")


# speedup vs baseline: 10.3638x; 10.3638x over previous
import functools

import jax
import jax.numpy as jnp
from jax import lax
from jax.experimental import pallas as pl
from jax.experimental.pallas import tpu as pltpu

ALPHA = 0.9984
R = 256


def _norm_h0_kernel(x_ref, win_ref, xn_ref, h0_ref):
    x = x_ref[...]
    n = jnp.sqrt(jnp.sum(x * x, axis=1, keepdims=True))
    xn_ref[...] = x / jnp.maximum(n, 1e-8)
    h0 = lax.dot_general(x, win_ref[...], (((1,), (0,)), ((), ())),
                         preferred_element_type=jnp.float32)
    h0_ref[...] = jnp.maximum(h0, 0.0)


def _sim_eps_kernel(n_real, k_top, xnb_ref, xnf_ref, sim_ref, eps_ref):
    i = pl.program_id(0)
    s = jnp.abs(lax.dot_general(xnb_ref[...], xnf_ref[...],
                                (((1,), (1,)), ((), ())),
                                preferred_element_type=jnp.float32))
    sim_ref[...] = s

    prev = jnp.full((R, 1), jnp.inf, jnp.float32)
    cnt = jnp.zeros((R, 1), jnp.float32)
    eps = jnp.zeros((R, 1), jnp.float32)
    done = jnp.zeros((R, 1), jnp.float32)
    for _ in range(k_top):
        masked = jnp.where(s < prev, s, -1.0)
        v = jnp.max(masked, axis=1, keepdims=True)
        cnt = cnt + jnp.sum(jnp.where(s == v, 1.0, 0.0), axis=1, keepdims=True)
        newly = jnp.where(cnt >= float(k_top), 1.0, 0.0) * (1.0 - done)
        eps = eps + newly * v
        done = jnp.minimum(done + newly, 1.0)
        prev = v
    rows = i * R + lax.broadcasted_iota(jnp.int32, (R, 1), 0)
    eps_ref[...] = jnp.where(rows < n_real, eps, jnp.inf)


def _sage_kernel(nb, sim_ref, eps_ref, hi_ref, hj_ref, wl_ref, wr_ref, b_ref,
                 out_ref, acc_ref, cnt_ref):
    i = pl.program_id(1)

    @pl.when(i == 0)
    def _():
        acc_ref[...] = jnp.zeros_like(acc_ref)
        cnt_ref[...] = jnp.zeros_like(cnt_ref)

    mask = (sim_ref[...] >= eps_ref[...]).astype(jnp.float32)
    acc_ref[...] += lax.dot_general(mask, hi_ref[...],
                                    (((0,), (0,)), ((), ())),
                                    preferred_element_type=jnp.float32)
    ones = jnp.ones((R, 1), jnp.float32)
    cnt_ref[...] += lax.dot_general(mask, ones, (((0,), (0,)), ((), ())),
                                    preferred_element_type=jnp.float32)

    @pl.when(i == nb - 1)
    def _():
        agg = acc_ref[...] / jnp.maximum(cnt_ref[...], 1.0)
        o = (lax.dot_general(agg, wl_ref[...], (((1,), (0,)), ((), ())),
                             preferred_element_type=jnp.float32)
             + lax.dot_general(hj_ref[...], wr_ref[...], (((1,), (0,)), ((), ())),
                               preferred_element_type=jnp.float32)
             + b_ref[...])
        out_ref[...] = jnp.maximum(o, 0.0)


def _proj_softmax_kernel(h_ref, w_ref, b_ref, out_ref):
    o = lax.dot_general(h_ref[...], w_ref[...], (((1,), (0,)), ((), ())),
                        preferred_element_type=jnp.float32) + b_ref[...]
    m = jnp.max(o, axis=1, keepdims=True)
    e = jnp.exp(o - m)
    out_ref[...] = e / jnp.sum(e, axis=1, keepdims=True)


def _sage_layer(sim, eps, h, wl, wr, b, np_, nb):
    hdim = h.shape[1]
    return pl.pallas_call(
        functools.partial(_sage_kernel, nb),
        grid=(nb, nb),
        in_specs=[
            pl.BlockSpec((R, R), lambda j, i: (i, j)),
            pl.BlockSpec((R, 1), lambda j, i: (i, 0)),
            pl.BlockSpec((R, hdim), lambda j, i: (i, 0)),
            pl.BlockSpec((R, hdim), lambda j, i: (j, 0)),
            pl.BlockSpec((hdim, hdim), lambda j, i: (0, 0)),
            pl.BlockSpec((hdim, hdim), lambda j, i: (0, 0)),
            pl.BlockSpec((1, hdim), lambda j, i: (0, 0)),
        ],
        out_specs=pl.BlockSpec((R, hdim), lambda j, i: (j, 0)),
        out_shape=jax.ShapeDtypeStruct((np_, hdim), jnp.float32),
        scratch_shapes=[pltpu.VMEM((R, hdim), jnp.float32),
                        pltpu.VMEM((R, 1), jnp.float32)],
        compiler_params=pltpu.CompilerParams(
            dimension_semantics=("parallel", "arbitrary")),
    )(sim, eps, h, h, wl, wr, b.reshape(1, hdim))


def kernel(x, W_in, Wl1, Wr1, b1, Wl2, Wr2, b2, W_out, b_out):
    n, d_in = x.shape
    hdim = W_in.shape[1]
    d_out = W_out.shape[1]
    np_ = ((n + R - 1) // R) * R
    nb = np_ // R
    k_top = n - int(round(ALPHA * (n - 1)))

    xp = jnp.pad(x, ((0, np_ - n), (0, 0)))

    xn, h0 = pl.pallas_call(
        _norm_h0_kernel,
        grid=(nb,),
        in_specs=[pl.BlockSpec((R, d_in), lambda i: (i, 0)),
                  pl.BlockSpec((d_in, hdim), lambda i: (0, 0))],
        out_specs=[pl.BlockSpec((R, d_in), lambda i: (i, 0)),
                   pl.BlockSpec((R, hdim), lambda i: (i, 0))],
        out_shape=[jax.ShapeDtypeStruct((np_, d_in), jnp.float32),
                   jax.ShapeDtypeStruct((np_, hdim), jnp.float32)],
    )(xp, W_in)

    sim, eps = pl.pallas_call(
        functools.partial(_sim_eps_kernel, n, k_top),
        grid=(nb,),
        in_specs=[pl.BlockSpec((R, d_in), lambda i: (i, 0)),
                  pl.BlockSpec((np_, d_in), lambda i: (0, 0))],
        out_specs=[pl.BlockSpec((R, np_), lambda i: (i, 0)),
                   pl.BlockSpec((R, 1), lambda i: (i, 0))],
        out_shape=[jax.ShapeDtypeStruct((np_, np_), jnp.float32),
                   jax.ShapeDtypeStruct((np_, 1), jnp.float32)],
    )(xn, xn)

    h1 = _sage_layer(sim, eps, h0, Wl1, Wr1, b1, np_, nb)
    h2 = _sage_layer(sim, eps, h1, Wl2, Wr2, b2, np_, nb)

    out = pl.pallas_call(
        _proj_softmax_kernel,
        grid=(nb,),
        in_specs=[pl.BlockSpec((R, hdim), lambda i: (i, 0)),
                  pl.BlockSpec((hdim, d_out), lambda i: (0, 0)),
                  pl.BlockSpec((1, d_out), lambda i: (0, 0))],
        out_specs=pl.BlockSpec((R, d_out), lambda i: (i, 0)),
        out_shape=jax.ShapeDtypeStruct((np_, d_out), jnp.float32),
    )(h2, W_out, b_out.reshape(1, d_out))

    return out[:n]


# speedup vs baseline: 11.0610x; 1.0673x over previous
import functools

import jax
import jax.numpy as jnp
from jax import lax
from jax.experimental import pallas as pl
from jax.experimental.pallas import tpu as pltpu

ALPHA = 0.9984
R = 256


def _norm_h0_kernel(x_ref, win_ref, xn_ref, h0_ref):
    x = x_ref[...]
    n = jnp.sqrt(jnp.sum(x * x, axis=1, keepdims=True))
    xn_ref[...] = x / jnp.maximum(n, 1e-8)
    h0 = lax.dot_general(x, win_ref[...], (((1,), (0,)), ((), ())),
                         preferred_element_type=jnp.float32)
    h0_ref[...] = jnp.maximum(h0, 0.0)


def _sim_mask_kernel(n_real, np_, k_top, xnb_ref, xnf_ref, mask_ref):
    i = pl.program_id(0)
    s = jnp.abs(lax.dot_general(xnb_ref[...], xnf_ref[...],
                                (((1,), (1,)), ((), ())),
                                preferred_element_type=jnp.float32))

    kf = float(k_top)
    prev = jnp.full((R, 1), jnp.inf, jnp.float32)
    eps = jnp.zeros((R, 1), jnp.float32)
    done = jnp.zeros((R, 1), jnp.float32)
    for k in range(k_top):
        lt = s < prev
        if k > 0:
            c_prev = float(np_) - jnp.sum(
                jnp.where(lt, 1.0, 0.0), axis=1, keepdims=True)
            newly = jnp.where(c_prev >= kf, 1.0, 0.0) * (1.0 - done)
            eps = eps + newly * prev
            done = jnp.minimum(done + newly, 1.0)
        masked = jnp.where(lt, s, -1.0)
        prev = jnp.max(masked, axis=1, keepdims=True)
    c_last = jnp.sum(jnp.where(s >= prev, 1.0, 0.0), axis=1, keepdims=True)
    newly = jnp.where(c_last >= kf, 1.0, 0.0) * (1.0 - done)
    eps = eps + newly * prev
    rows = i * R + lax.broadcasted_iota(jnp.int32, (R, 1), 0)
    eps = jnp.where(rows < n_real, eps, jnp.inf)
    mask_ref[...] = (s >= eps).astype(jnp.int8)


def _sage_kernel(nb, mask_ref, hi_ref, hj_ref, wl_ref, wr_ref, b_ref,
                 out_ref, acc_ref, cnt_ref):
    i = pl.program_id(1)

    @pl.when(i == 0)
    def _():
        acc_ref[...] = jnp.zeros_like(acc_ref)
        cnt_ref[...] = jnp.zeros_like(cnt_ref)

    mask = mask_ref[...].astype(jnp.float32)
    acc_ref[...] += lax.dot_general(mask, hi_ref[...],
                                    (((0,), (0,)), ((), ())),
                                    preferred_element_type=jnp.float32)
    ones = jnp.ones((R, 1), jnp.float32)
    cnt_ref[...] += lax.dot_general(mask, ones, (((0,), (0,)), ((), ())),
                                    preferred_element_type=jnp.float32)

    @pl.when(i == nb - 1)
    def _():
        agg = acc_ref[...] / jnp.maximum(cnt_ref[...], 1.0)
        o = (lax.dot_general(agg, wl_ref[...], (((1,), (0,)), ((), ())),
                             preferred_element_type=jnp.float32)
             + lax.dot_general(hj_ref[...], wr_ref[...], (((1,), (0,)), ((), ())),
                               preferred_element_type=jnp.float32)
             + b_ref[...])
        out_ref[...] = jnp.maximum(o, 0.0)


def _proj_softmax_kernel(h_ref, w_ref, b_ref, out_ref):
    o = lax.dot_general(h_ref[...], w_ref[...], (((1,), (0,)), ((), ())),
                        preferred_element_type=jnp.float32) + b_ref[...]
    m = jnp.max(o, axis=1, keepdims=True)
    e = jnp.exp(o - m)
    out_ref[...] = e / jnp.sum(e, axis=1, keepdims=True)


def _sage_layer(mask, h, wl, wr, b, np_, nb):
    hdim = h.shape[1]
    return pl.pallas_call(
        functools.partial(_sage_kernel, nb),
        grid=(nb, nb),
        in_specs=[
            pl.BlockSpec((R, R), lambda j, i: (i, j)),
            pl.BlockSpec((R, hdim), lambda j, i: (i, 0)),
            pl.BlockSpec((R, hdim), lambda j, i: (j, 0)),
            pl.BlockSpec((hdim, hdim), lambda j, i: (0, 0)),
            pl.BlockSpec((hdim, hdim), lambda j, i: (0, 0)),
            pl.BlockSpec((1, hdim), lambda j, i: (0, 0)),
        ],
        out_specs=pl.BlockSpec((R, hdim), lambda j, i: (j, 0)),
        out_shape=jax.ShapeDtypeStruct((np_, hdim), jnp.float32),
        scratch_shapes=[pltpu.VMEM((R, hdim), jnp.float32),
                        pltpu.VMEM((R, 1), jnp.float32)],
        compiler_params=pltpu.CompilerParams(
            dimension_semantics=("parallel", "arbitrary")),
    )(mask, h, h, wl, wr, b.reshape(1, hdim))


def kernel(x, W_in, Wl1, Wr1, b1, Wl2, Wr2, b2, W_out, b_out):
    n, d_in = x.shape
    hdim = W_in.shape[1]
    d_out = W_out.shape[1]
    np_ = ((n + R - 1) // R) * R
    nb = np_ // R
    k_top = n - int(round(ALPHA * (n - 1)))

    xp = jnp.pad(x, ((0, np_ - n), (0, 0)))

    xn, h0 = pl.pallas_call(
        _norm_h0_kernel,
        grid=(nb,),
        in_specs=[pl.BlockSpec((R, d_in), lambda i: (i, 0)),
                  pl.BlockSpec((d_in, hdim), lambda i: (0, 0))],
        out_specs=[pl.BlockSpec((R, d_in), lambda i: (i, 0)),
                   pl.BlockSpec((R, hdim), lambda i: (i, 0))],
        out_shape=[jax.ShapeDtypeStruct((np_, d_in), jnp.float32),
                   jax.ShapeDtypeStruct((np_, hdim), jnp.float32)],
    )(xp, W_in)

    mask = pl.pallas_call(
        functools.partial(_sim_mask_kernel, n, np_, k_top),
        grid=(nb,),
        in_specs=[pl.BlockSpec((R, d_in), lambda i: (i, 0)),
                  pl.BlockSpec((np_, d_in), lambda i: (0, 0))],
        out_specs=pl.BlockSpec((R, np_), lambda i: (i, 0)),
        out_shape=jax.ShapeDtypeStruct((np_, np_), jnp.int8),
    )(xn, xn)

    h1 = _sage_layer(mask, h0, Wl1, Wr1, b1, np_, nb)
    h2 = _sage_layer(mask, h1, Wl2, Wr2, b2, np_, nb)

    out = pl.pallas_call(
        _proj_softmax_kernel,
        grid=(nb,),
        in_specs=[pl.BlockSpec((R, hdim), lambda i: (i, 0)),
                  pl.BlockSpec((hdim, d_out), lambda i: (0, 0)),
                  pl.BlockSpec((1, d_out), lambda i: (0, 0))],
        out_specs=pl.BlockSpec((R, d_out), lambda i: (i, 0)),
        out_shape=jax.ShapeDtypeStruct((np_, d_out), jnp.float32),
    )(h2, W_out, b_out.reshape(1, d_out))

    return out[:n]


# speedup vs baseline: 18.8345x; 1.7028x over previous
import functools

import jax
import jax.numpy as jnp
from jax import lax
from jax.experimental import pallas as pl
from jax.experimental.pallas import tpu as pltpu

ALPHA = 0.9984
R = 256


def _norm_h0_kernel(x_ref, win_ref, xn_ref, h0_ref):
    x = x_ref[...]
    n = jnp.sqrt(jnp.sum(x * x, axis=1, keepdims=True))
    xn_ref[...] = x / jnp.maximum(n, 1e-8)
    h0 = lax.dot_general(x, win_ref[...], (((1,), (0,)), ((), ())),
                         preferred_element_type=jnp.float32)
    h0_ref[...] = jnp.maximum(h0, 0.0)


def _sim_mask_kernel(n_real, np_, nb, k_top, xnb_ref, xnf_ref, mask_ref,
                     cnt_ref, cacc_ref):
    i = pl.program_id(0)
    s = jnp.abs(lax.dot_general(xnb_ref[...], xnf_ref[...],
                                (((1,), (1,)), ((), ())),
                                preferred_element_type=jnp.float32))

    kf = float(k_top)
    prev = jnp.full((R, 1), jnp.inf, jnp.float32)
    eps = jnp.zeros((R, 1), jnp.float32)
    done = jnp.zeros((R, 1), jnp.float32)
    for k in range(k_top):
        lt = s < prev
        if k > 0:
            c_prev = float(np_) - jnp.sum(
                jnp.where(lt, 1.0, 0.0), axis=1, keepdims=True)
            newly = jnp.where(c_prev >= kf, 1.0, 0.0) * (1.0 - done)
            eps = eps + newly * prev
            done = jnp.minimum(done + newly, 1.0)
        masked = jnp.where(lt, s, -1.0)
        prev = jnp.max(masked, axis=1, keepdims=True)
    c_last = jnp.sum(jnp.where(s >= prev, 1.0, 0.0), axis=1, keepdims=True)
    newly = jnp.where(c_last >= kf, 1.0, 0.0) * (1.0 - done)
    eps = eps + newly * prev
    rows = i * R + lax.broadcasted_iota(jnp.int32, (R, 1), 0)
    eps = jnp.where(rows < n_real, eps, jnp.inf)
    maskf = jnp.where(s >= eps, 1.0, 0.0)
    mask_ref[...] = maskf.astype(jnp.int8)

    @pl.when(i == 0)
    def _():
        cacc_ref[...] = jnp.zeros_like(cacc_ref)

    cacc_ref[...] += jnp.sum(maskf, axis=0, keepdims=True)

    @pl.when(i == nb - 1)
    def _():
        cnt_ref[...] = cacc_ref[...]


def _sage_kernel(nb, mask_ref, cnt_ref, hi_ref, hj_ref, wl_ref, wr_ref, b_ref,
                 out_ref, acc_ref):
    i = pl.program_id(1)

    @pl.when(i == 0)
    def _():
        acc_ref[...] = jnp.zeros_like(acc_ref)

    mask = mask_ref[...].astype(jnp.float32)
    acc_ref[...] += lax.dot_general(mask, hi_ref[...],
                                    (((0,), (0,)), ((), ())),
                                    preferred_element_type=jnp.float32)

    @pl.when(i == nb - 1)
    def _():
        agg = acc_ref[...] / jnp.maximum(cnt_ref[...], 1.0)
        o = (lax.dot_general(agg, wl_ref[...], (((1,), (0,)), ((), ())),
                             preferred_element_type=jnp.float32)
             + lax.dot_general(hj_ref[...], wr_ref[...], (((1,), (0,)), ((), ())),
                               preferred_element_type=jnp.float32)
             + b_ref[...])
        out_ref[...] = jnp.maximum(o, 0.0)


def _proj_softmax_kernel(h_ref, w_ref, b_ref, out_ref):
    o = lax.dot_general(h_ref[...], w_ref[...], (((1,), (0,)), ((), ())),
                        preferred_element_type=jnp.float32) + b_ref[...]
    m = jnp.max(o, axis=1, keepdims=True)
    e = jnp.exp(o - m)
    out_ref[...] = e / jnp.sum(e, axis=1, keepdims=True)


RJ = 2048


def _sage_layer(mask, cnt_col, h, wl, wr, b, np_, nb):
    hdim = h.shape[1]
    rj = min(RJ, np_)
    nj = np_ // rj
    return pl.pallas_call(
        functools.partial(_sage_kernel, nb),
        grid=(nj, nb),
        in_specs=[
            pl.BlockSpec((R, rj), lambda j, i: (i, j)),
            pl.BlockSpec((rj, 1), lambda j, i: (j, 0)),
            pl.BlockSpec((R, hdim), lambda j, i: (i, 0)),
            pl.BlockSpec((rj, hdim), lambda j, i: (j, 0)),
            pl.BlockSpec((hdim, hdim), lambda j, i: (0, 0)),
            pl.BlockSpec((hdim, hdim), lambda j, i: (0, 0)),
            pl.BlockSpec((1, hdim), lambda j, i: (0, 0)),
        ],
        out_specs=pl.BlockSpec((rj, hdim), lambda j, i: (j, 0)),
        out_shape=jax.ShapeDtypeStruct((np_, hdim), jnp.float32),
        scratch_shapes=[pltpu.VMEM((rj, hdim), jnp.float32)],
        compiler_params=pltpu.CompilerParams(
            dimension_semantics=("parallel", "arbitrary")),
    )(mask, cnt_col, h, h, wl, wr, b.reshape(1, hdim))


def kernel(x, W_in, Wl1, Wr1, b1, Wl2, Wr2, b2, W_out, b_out):
    n, d_in = x.shape
    hdim = W_in.shape[1]
    d_out = W_out.shape[1]
    np_ = ((n + R - 1) // R) * R
    nb = np_ // R
    k_top = n - int(round(ALPHA * (n - 1)))

    xp = jnp.pad(x, ((0, np_ - n), (0, 0)))

    xn, h0 = pl.pallas_call(
        _norm_h0_kernel,
        grid=(nb,),
        in_specs=[pl.BlockSpec((R, d_in), lambda i: (i, 0)),
                  pl.BlockSpec((d_in, hdim), lambda i: (0, 0))],
        out_specs=[pl.BlockSpec((R, d_in), lambda i: (i, 0)),
                   pl.BlockSpec((R, hdim), lambda i: (i, 0))],
        out_shape=[jax.ShapeDtypeStruct((np_, d_in), jnp.float32),
                   jax.ShapeDtypeStruct((np_, hdim), jnp.float32)],
    )(xp, W_in)

    mask, cnt_row = pl.pallas_call(
        functools.partial(_sim_mask_kernel, n, np_, nb, k_top),
        grid=(nb,),
        in_specs=[pl.BlockSpec((R, d_in), lambda i: (i, 0)),
                  pl.BlockSpec((np_, d_in), lambda i: (0, 0))],
        out_specs=[pl.BlockSpec((R, np_), lambda i: (i, 0)),
                   pl.BlockSpec((1, np_), lambda i: (0, 0))],
        out_shape=[jax.ShapeDtypeStruct((np_, np_), jnp.int8),
                   jax.ShapeDtypeStruct((1, np_), jnp.float32)],
        scratch_shapes=[pltpu.VMEM((1, np_), jnp.float32)],
    )(xn, xn)
    cnt_col = cnt_row.reshape(np_, 1)

    h1 = _sage_layer(mask, cnt_col, h0, Wl1, Wr1, b1, np_, nb)
    h2 = _sage_layer(mask, cnt_col, h1, Wl2, Wr2, b2, np_, nb)

    out = pl.pallas_call(
        _proj_softmax_kernel,
        grid=(nb,),
        in_specs=[pl.BlockSpec((R, hdim), lambda i: (i, 0)),
                  pl.BlockSpec((hdim, d_out), lambda i: (0, 0)),
                  pl.BlockSpec((1, d_out), lambda i: (0, 0))],
        out_specs=pl.BlockSpec((R, d_out), lambda i: (i, 0)),
        out_shape=jax.ShapeDtypeStruct((np_, d_out), jnp.float32),
    )(h2, W_out, b_out.reshape(1, d_out))

    return out[:n]


# speedup vs baseline: 31.8241x; 1.6897x over previous
import functools

import jax
import jax.numpy as jnp
from jax import lax
from jax.experimental import pallas as pl
from jax.experimental.pallas import tpu as pltpu

ALPHA = 0.9984
R = 256


def _norm_h0_kernel(x_ref, win_ref, xn_ref, h0_ref):
    x = x_ref[...]
    n = jnp.sqrt(jnp.sum(x * x, axis=1, keepdims=True))
    xn_ref[...] = x / jnp.maximum(n, 1e-8)
    h0 = lax.dot_general(x, win_ref[...], (((1,), (0,)), ((), ())),
                         preferred_element_type=jnp.float32)
    h0_ref[...] = jnp.maximum(h0, 0.0)


def _sim_mask_kernel(n_real, np_, nb, k_top, xnb_ref, xnf_ref, mask_ref,
                     cnt_ref, cacc_ref, eps_ref):
    i = pl.program_id(0)
    s = jnp.abs(lax.dot_general(xnb_ref[...], xnf_ref[...],
                                (((1,), (1,)), ((), ())),
                                preferred_element_type=jnp.float32))
    rows = i * R + lax.broadcasted_iota(jnp.int32, (R, 1), 0)
    real = rows < n_real
    kf = float(k_top)

    prev = jnp.full((R, 1), jnp.inf, jnp.float32)
    for _ in range(k_top):
        masked = jnp.where(s < prev, s, -1.0)
        prev = jnp.max(masked, axis=1, keepdims=True)
    c_last = jnp.sum(jnp.where(s >= prev, 1.0, 0.0), axis=1, keepdims=True)
    eps_ref[...] = jnp.where(real, prev, jnp.inf)
    dup = jnp.where(jnp.logical_and(c_last != kf, real), 1.0, 0.0)

    @pl.when(jnp.max(dup) > 0.0)
    def _():
        prev2 = jnp.full((R, 1), jnp.inf, jnp.float32)
        eps2 = jnp.zeros((R, 1), jnp.float32)
        done = jnp.zeros((R, 1), jnp.float32)
        for _ in range(k_top):
            masked = jnp.where(s < prev2, s, -1.0)
            v = jnp.max(masked, axis=1, keepdims=True)
            c = jnp.sum(jnp.where(s >= v, 1.0, 0.0), axis=1, keepdims=True)
            newly = jnp.where(c >= kf, 1.0, 0.0) * (1.0 - done)
            eps2 = eps2 + newly * v
            done = jnp.minimum(done + newly, 1.0)
            prev2 = v
        eps_ref[...] = jnp.where(dup > 0.0, eps2, eps_ref[...])

    eps = eps_ref[...]
    maskf = jnp.where(s >= eps, 1.0, 0.0)
    mask_ref[...] = maskf.astype(jnp.int8)

    @pl.when(i == 0)
    def _():
        cacc_ref[...] = jnp.zeros_like(cacc_ref)

    cacc_ref[...] += jnp.sum(maskf, axis=0, keepdims=True)

    @pl.when(i == nb - 1)
    def _():
        cnt_ref[...] = cacc_ref[...]


def _sage_kernel(nb, mask_ref, cnt_ref, hi_ref, hj_ref, wl_ref, wr_ref, b_ref,
                 out_ref, acc_ref):
    i = pl.program_id(1)

    @pl.when(i == 0)
    def _():
        acc_ref[...] = jnp.zeros_like(acc_ref)

    mask = mask_ref[...].astype(jnp.float32)
    acc_ref[...] += lax.dot_general(mask, hi_ref[...],
                                    (((0,), (0,)), ((), ())),
                                    preferred_element_type=jnp.float32)

    @pl.when(i == nb - 1)
    def _():
        agg = acc_ref[...] / jnp.maximum(cnt_ref[...], 1.0)
        o = (lax.dot_general(agg, wl_ref[...], (((1,), (0,)), ((), ())),
                             preferred_element_type=jnp.float32)
             + lax.dot_general(hj_ref[...], wr_ref[...], (((1,), (0,)), ((), ())),
                               preferred_element_type=jnp.float32)
             + b_ref[...])
        out_ref[...] = jnp.maximum(o, 0.0)


def _proj_softmax_kernel(h_ref, w_ref, b_ref, out_ref):
    o = lax.dot_general(h_ref[...], w_ref[...], (((1,), (0,)), ((), ())),
                        preferred_element_type=jnp.float32) + b_ref[...]
    m = jnp.max(o, axis=1, keepdims=True)
    e = jnp.exp(o - m)
    out_ref[...] = e / jnp.sum(e, axis=1, keepdims=True)


RJ = 2048


def _sage_layer(mask, cnt_col, h, wl, wr, b, np_, nb):
    hdim = h.shape[1]
    rj = min(RJ, np_)
    nj = np_ // rj
    return pl.pallas_call(
        functools.partial(_sage_kernel, nb),
        grid=(nj, nb),
        in_specs=[
            pl.BlockSpec((R, rj), lambda j, i: (i, j)),
            pl.BlockSpec((rj, 1), lambda j, i: (j, 0)),
            pl.BlockSpec((R, hdim), lambda j, i: (i, 0)),
            pl.BlockSpec((rj, hdim), lambda j, i: (j, 0)),
            pl.BlockSpec((hdim, hdim), lambda j, i: (0, 0)),
            pl.BlockSpec((hdim, hdim), lambda j, i: (0, 0)),
            pl.BlockSpec((1, hdim), lambda j, i: (0, 0)),
        ],
        out_specs=pl.BlockSpec((rj, hdim), lambda j, i: (j, 0)),
        out_shape=jax.ShapeDtypeStruct((np_, hdim), jnp.float32),
        scratch_shapes=[pltpu.VMEM((rj, hdim), jnp.float32)],
        compiler_params=pltpu.CompilerParams(
            dimension_semantics=("parallel", "arbitrary")),
    )(mask, cnt_col, h, h, wl, wr, b.reshape(1, hdim))


def kernel(x, W_in, Wl1, Wr1, b1, Wl2, Wr2, b2, W_out, b_out):
    n, d_in = x.shape
    hdim = W_in.shape[1]
    d_out = W_out.shape[1]
    np_ = ((n + R - 1) // R) * R
    nb = np_ // R
    k_top = n - int(round(ALPHA * (n - 1)))

    xp = jnp.pad(x, ((0, np_ - n), (0, 0)))

    xn, h0 = pl.pallas_call(
        _norm_h0_kernel,
        grid=(nb,),
        in_specs=[pl.BlockSpec((R, d_in), lambda i: (i, 0)),
                  pl.BlockSpec((d_in, hdim), lambda i: (0, 0))],
        out_specs=[pl.BlockSpec((R, d_in), lambda i: (i, 0)),
                   pl.BlockSpec((R, hdim), lambda i: (i, 0))],
        out_shape=[jax.ShapeDtypeStruct((np_, d_in), jnp.float32),
                   jax.ShapeDtypeStruct((np_, hdim), jnp.float32)],
    )(xp, W_in)

    mask, cnt_row = pl.pallas_call(
        functools.partial(_sim_mask_kernel, n, np_, nb, k_top),
        grid=(nb,),
        in_specs=[pl.BlockSpec((R, d_in), lambda i: (i, 0)),
                  pl.BlockSpec((np_, d_in), lambda i: (0, 0))],
        out_specs=[pl.BlockSpec((R, np_), lambda i: (i, 0)),
                   pl.BlockSpec((1, np_), lambda i: (0, 0))],
        out_shape=[jax.ShapeDtypeStruct((np_, np_), jnp.int8),
                   jax.ShapeDtypeStruct((1, np_), jnp.float32)],
        scratch_shapes=[pltpu.VMEM((1, np_), jnp.float32),
                        pltpu.VMEM((R, 1), jnp.float32)],
    )(xn, xn)
    cnt_col = cnt_row.reshape(np_, 1)

    h1 = _sage_layer(mask, cnt_col, h0, Wl1, Wr1, b1, np_, nb)
    h2 = _sage_layer(mask, cnt_col, h1, Wl2, Wr2, b2, np_, nb)

    out = pl.pallas_call(
        _proj_softmax_kernel,
        grid=(nb,),
        in_specs=[pl.BlockSpec((R, hdim), lambda i: (i, 0)),
                  pl.BlockSpec((hdim, d_out), lambda i: (0, 0)),
                  pl.BlockSpec((1, d_out), lambda i: (0, 0))],
        out_specs=pl.BlockSpec((R, d_out), lambda i: (i, 0)),
        out_shape=jax.ShapeDtypeStruct((np_, d_out), jnp.float32),
    )(h2, W_out, b_out.reshape(1, d_out))

    return out[:n]
